# baseline (speedup 1.0000x reference)
import functools

import jax
import jax.numpy as jnp
import numpy as np
from jax import lax
from jax.experimental import pallas as pl
from jax.experimental.pallas import tpu as pltpu
from jax.experimental.pallas import tpu_sc as plsc

_MU = 100.0
_M = 256.0
_NB = 1024
_SHIFT = 32000
_SEP = _NB + _SHIFT
_KS = float(512.0 / np.log2(_M * _MU + 1.0))

_C0 = 1.4426156832028272
_C1 = -0.7170639321925276
_C2 = 0.44227417973723016
_C3 = -0.2277126446965988
_C4 = 0.059945587461487176

_RC = 800
_TW = _RC * 16
_AW = _RC * 8
_OW = _RC * 25
_NSUB = 32


def _tok16(x):
    y = jnp.abs(x) * jnp.float32(_MU) + jnp.float32(1.0)
    b = plsc.bitcast(y, jnp.int32)
    e = (b >> 23) - 127
    m = plsc.bitcast((b & 0x007FFFFF) | 0x3F800000, jnp.float32)
    t = m - jnp.float32(1.0)
    q = jnp.float32(_C4)
    q = q * t + jnp.float32(_C3)
    q = q * t + jnp.float32(_C2)
    q = q * t + jnp.float32(_C1)
    q = q * t + jnp.float32(_C0)
    l2 = e.astype(jnp.float32) + q * t
    ks = jnp.where(x < jnp.float32(0.0), jnp.float32(-_KS), jnp.float32(_KS))
    v = l2 * ks + jnp.float32(512.0)
    v = jnp.minimum(jnp.maximum(v, jnp.float32(0.0)), jnp.float32(1023.0))
    return v.astype(jnp.int32) + _SHIFT


def _sc_body(nchunk, t_hbm, a_hbm, o_hbm, t_buf, a_buf, o_buf):
    wid = lax.axis_index("s") * 2 + lax.axis_index("c")
    iota = lax.iota(jnp.int32, 16)
    pat_a = (iota >> 3) * 25 + (iota & 7) + 17
    pat_s = iota * 25 + 16
    sep = jnp.full((16,), _SEP, jnp.int32)
    nmine = (nchunk - 1 - wid) // _NSUB + 1

    def chunk_body(k, _):
        c = wid + k * _NSUB
        pltpu.sync_copy(t_hbm.at[pl.ds(c * _TW, _TW)], t_buf)
        pltpu.sync_copy(a_hbm.at[pl.ds(c * _AW, _AW)], a_buf)

        def t_loop(i, idx):
            plsc.store_scatter(o_buf, [idx], _tok16(t_buf[pl.ds(i * 16, 16)]))
            return idx + 25

        lax.fori_loop(0, _RC, t_loop, iota)

        def a_loop(i, idx):
            plsc.store_scatter(o_buf, [idx], _tok16(a_buf[pl.ds(i * 16, 16)]))
            return idx + 50

        lax.fori_loop(0, _RC // 2, a_loop, pat_a)

        def s_loop(i, idx):
            plsc.store_scatter(o_buf, [idx], sep)
            return idx + 400

        lax.fori_loop(0, _RC // 16, s_loop, pat_s)

        pltpu.sync_copy(o_buf, o_hbm.at[pl.ds(c * _OW, _OW)])
        return 0

    lax.fori_loop(0, nmine, chunk_body, 0)


@jax.jit
def kernel(tensors, actions):
    n = tensors.shape[0]
    assert n % _RC == 0
    nchunk = n // _RC
    mesh = plsc.VectorSubcoreMesh(core_axis_name="c", subcore_axis_name="s")
    run = pl.kernel(
        functools.partial(_sc_body, nchunk),
        out_type=jax.ShapeDtypeStruct((n * 25,), jnp.int32),
        mesh=mesh,
        compiler_params=pltpu.CompilerParams(needs_layout_passes=False),
        scratch_types=[
            pltpu.VMEM((_TW,), jnp.float32),
            pltpu.VMEM((_AW,), jnp.float32),
            pltpu.VMEM((_OW,), jnp.int32),
        ],
    )
    out = run(tensors.reshape(-1), actions.reshape(-1))
    return out.reshape(n, 25)


# speedup vs baseline: 1.7325x; 1.7325x over previous
import functools

import jax
import jax.numpy as jnp
import numpy as np
from jax import lax
from jax.experimental import pallas as pl
from jax.experimental.pallas import tpu as pltpu
from jax.experimental.pallas import tpu_sc as plsc

_MU = 100.0
_M = 256.0
_NB = 1024
_SHIFT = 32000
_SEP = _NB + _SHIFT
_KS = float(512.0 / np.log2(_M * _MU + 1.0))

_C0 = 1.4426156832028272
_C1 = -0.7170639321925276
_C2 = 0.44227417973723016
_C3 = -0.2277126446965988
_C4 = 0.059945587461487176

_RC = 800
_NSUB = 32


def _tok16(x):
    y = jnp.abs(x) * jnp.float32(_MU) + jnp.float32(1.0)
    b = plsc.bitcast(y, jnp.int32)
    e = (b >> 23) - 127
    m = plsc.bitcast((b & 0x007FFFFF) | 0x3F800000, jnp.float32)
    t = m - jnp.float32(1.0)
    q = jnp.float32(_C4)
    q = q * t + jnp.float32(_C3)
    q = q * t + jnp.float32(_C2)
    q = q * t + jnp.float32(_C1)
    q = q * t + jnp.float32(_C0)
    l2 = e.astype(jnp.float32) + q * t
    ks = jnp.where(x < jnp.float32(0.0), jnp.float32(-_KS), jnp.float32(_KS))
    v = l2 * ks + jnp.float32(512.0 + _SHIFT)
    v = jnp.minimum(jnp.maximum(v, jnp.float32(_SHIFT)),
                    jnp.float32(_SHIFT + _NB - 1))
    return v.astype(jnp.int32)


def _sc_body(nchunk, t_hbm, a_hbm, o_hbm, t_buf, a_buf, o_buf):
    wid = lax.axis_index("s") * 2 + lax.axis_index("c")
    iota = lax.iota(jnp.int32, 16)
    arow = iota >> 3
    acol = iota & 7
    ocol = acol + 17
    sep = jnp.full((16,), _SEP, jnp.int32)
    col16 = jnp.full((16,), 16, jnp.int32)
    nmine = (nchunk - 1 - wid) // _NSUB + 1

    def chunk_body(k, _):
        r0 = (wid + k * _NSUB) * _RC
        pltpu.sync_copy(t_hbm.at[pl.ds(r0, _RC), :], t_buf)
        pltpu.sync_copy(a_hbm.at[pl.ds(r0, _RC), :], a_buf)

        @plsc.parallel_loop(0, _RC, unroll=8)
        def t_loop(i):
            o_buf[i, pl.ds(0, 16)] = _tok16(t_buf[i, :])

        @plsc.parallel_loop(0, _RC // 2, unroll=8)
        def a_loop(i):
            row = 2 * i + arow
            x = plsc.load_gather(a_buf, [row, acol])
            plsc.store_scatter(o_buf, [row, ocol], _tok16(x))

        @plsc.parallel_loop(0, _RC // 16, unroll=4)
        def s_loop(i):
            plsc.store_scatter(o_buf, [16 * i + iota, col16], sep)

        pltpu.sync_copy(o_buf, o_hbm.at[pl.ds(r0, _RC), :])
        return 0

    lax.fori_loop(0, nmine, chunk_body, 0)


@jax.jit
def kernel(tensors, actions):
    n = tensors.shape[0]
    assert n % _RC == 0
    nchunk = n // _RC
    mesh = plsc.VectorSubcoreMesh(core_axis_name="c", subcore_axis_name="s")
    run = pl.kernel(
        functools.partial(_sc_body, nchunk),
        out_type=jax.ShapeDtypeStruct((n, 25), jnp.int32),
        mesh=mesh,
        compiler_params=pltpu.CompilerParams(needs_layout_passes=False,
                                             use_tc_tiling_on_sc=False),
        scratch_types=[
            pltpu.VMEM((_RC, 16), jnp.float32),
            pltpu.VMEM((_RC, 8), jnp.float32),
            pltpu.VMEM((_RC, 25), jnp.int32),
        ],
    )
    return run(tensors, actions)


# speedup vs baseline: 6.5110x; 3.7581x over previous
import functools

import jax
import jax.numpy as jnp
import numpy as np
from jax import lax
from jax.experimental import pallas as pl
from jax.experimental.pallas import tpu as pltpu
from jax.experimental.pallas import tpu_sc as plsc

_MU = 100.0
_M = 256.0
_NB = 1024
_SHIFT = 32000
_SEP = _NB + _SHIFT
_KS = float(512.0 / np.log2(_M * _MU + 1.0))

_C0 = 1.4426156832028272
_C1 = -0.7170639321925276
_C2 = 0.44227417973723016
_C3 = -0.2277126446965988
_C4 = 0.059945587461487176

_CW = 1024
_NSUB = 32


def _tok16(x):
    y = jnp.abs(x) * jnp.float32(_MU) + jnp.float32(1.0)
    b = plsc.bitcast(y, jnp.int32)
    e = (b >> 23) - 127
    m = plsc.bitcast((b & 0x007FFFFF) | 0x3F800000, jnp.float32)
    t = m - jnp.float32(1.0)
    q = jnp.float32(_C4)
    q = q * t + jnp.float32(_C3)
    q = q * t + jnp.float32(_C2)
    q = q * t + jnp.float32(_C1)
    q = q * t + jnp.float32(_C0)
    l2 = e.astype(jnp.float32) + q * t
    ks = jnp.where(x < jnp.float32(0.0), jnp.float32(-_KS), jnp.float32(_KS))
    v = l2 * ks + jnp.float32(512.0 + _SHIFT)
    v = jnp.minimum(jnp.maximum(v, jnp.float32(_SHIFT)),
                    jnp.float32(_SHIFT + _NB - 1))
    return v.astype(jnp.int32)


def _sc_body(nfull, remc0, remw, t_hbm, a_hbm, o_hbm, t_buf, a_buf, o_buf):
    wid = lax.axis_index("s") * 2 + lax.axis_index("c")
    sep = jnp.full((16,), _SEP, jnp.int32)
    nmine = (nfull - 1 - wid) // _NSUB + 1

    def process(c0, w):
        pltpu.sync_copy(t_hbm.at[:, pl.ds(c0, w)], t_buf.at[:, pl.ds(0, w)])
        pltpu.sync_copy(a_hbm.at[:, pl.ds(c0, w)], a_buf.at[:, pl.ds(0, w)])

        @plsc.parallel_loop(0, w // 16)
        def body(j):
            c = j * 16
            for r in range(16):
                o_buf[r, pl.ds(c, 16)] = _tok16(t_buf[r, pl.ds(c, 16)])
            for r in range(8):
                o_buf[17 + r, pl.ds(c, 16)] = _tok16(a_buf[r, pl.ds(c, 16)])
            o_buf[16, pl.ds(c, 16)] = sep

        pltpu.sync_copy(o_buf.at[:, pl.ds(0, w)], o_hbm.at[:, pl.ds(c0, w)])

    def chunk_body(k, _):
        process((wid + k * _NSUB) * _CW, _CW)
        return 0

    lax.fori_loop(0, nmine, chunk_body, 0)

    if remw:
        @pl.when(wid == 8)
        def _():
            process(remc0, remw)


def _tok_ref(x):
    mu = jnp.sign(x) * jnp.log(jnp.abs(x) * _MU + 1.0) / np.log(_M * _MU + 1.0)
    v = jnp.floor((jnp.clip(mu, -1.0, 1.0) + 1.0) * (_NB / 2))
    return jnp.clip(v, 0.0, _NB - 1).astype(jnp.int32) + _SHIFT


@jax.jit
def kernel(tensors, actions):
    n = tensors.shape[0]
    nmain = (n // 128) * 128
    nfull = nmain // _CW
    remc0 = nfull * _CW
    remw = nmain - remc0
    mesh = plsc.VectorSubcoreMesh(core_axis_name="c", subcore_axis_name="s")
    run = pl.kernel(
        functools.partial(_sc_body, nfull, remc0, remw),
        out_type=jax.ShapeDtypeStruct((25, n), jnp.int32),
        mesh=mesh,
        compiler_params=pltpu.CompilerParams(needs_layout_passes=False,
                                             use_tc_tiling_on_sc=True),
        scratch_types=[
            pltpu.VMEM((16, _CW), jnp.float32),
            pltpu.VMEM((8, _CW), jnp.float32),
            pltpu.VMEM((25, _CW), jnp.int32),
        ],
    )
    out = run(tensors.T, actions.T).T
    if nmain == n:
        return out
    tt = _tok_ref(tensors[nmain:])
    at = _tok_ref(actions[nmain:])
    sepcol = jnp.full((n - nmain, 1), _SEP, jnp.int32)
    tail = jnp.concatenate([tt, sepcol, at], axis=1)
    return lax.dynamic_update_slice(out, tail, (nmain, 0))


# speedup vs baseline: 8.3387x; 1.2807x over previous
import functools

import jax
import jax.numpy as jnp
import numpy as np
from jax import lax
from jax.experimental import pallas as pl
from jax.experimental.pallas import tpu as pltpu
from jax.experimental.pallas import tpu_sc as plsc

_MU = 100.0
_M = 256.0
_NB = 1024
_SHIFT = 32000
_SEP = _NB + _SHIFT
_KS = float(512.0 / np.log2(_M * _MU + 1.0))

_C0 = 1.4426156832028272
_C1 = -0.7170639321925276
_C2 = 0.44227417973723016
_C3 = -0.2277126446965988
_C4 = 0.059945587461487176

_CW = 1024
_NSUB = 32


def _tok16(x):
    y = jnp.abs(x) * jnp.float32(_MU) + jnp.float32(1.0)
    b = plsc.bitcast(y, jnp.int32)
    e = (b >> 23) - 127
    m = plsc.bitcast((b & 0x007FFFFF) | 0x3F800000, jnp.float32)
    t = m - jnp.float32(1.0)
    q = jnp.float32(_C4)
    q = q * t + jnp.float32(_C3)
    q = q * t + jnp.float32(_C2)
    q = q * t + jnp.float32(_C1)
    q = q * t + jnp.float32(_C0)
    l2 = e.astype(jnp.float32) + q * t
    ks = jnp.where(x < jnp.float32(0.0), jnp.float32(-_KS), jnp.float32(_KS))
    v = l2 * ks + jnp.float32(512.0 + _SHIFT)
    v = jnp.minimum(jnp.maximum(v, jnp.float32(_SHIFT)),
                    jnp.float32(_SHIFT + _NB - 1))
    return v.astype(jnp.int32)


def _sc_body(nfull, remc0, remw, t_hbm, a_hbm, o_hbm,
             t0, t1, a0, a1, o0, o1, si0, si1, so0, so1):
    wid = lax.axis_index("s") * 2 + lax.axis_index("c")
    sep = jnp.full((16,), _SEP, jnp.int32)
    nmine = (nfull - 1 - wid) // _NSUB + 1
    tb, ab, ob = (t0, t1), (a0, a1), (o0, o1)
    sin, sout = (si0, si1), (so0, so1)

    def c0_of(k):
        return (wid + k * _NSUB) * _CW

    def start_in(k, b):
        c0 = c0_of(k)
        pltpu.async_copy(t_hbm.at[:, pl.ds(c0, _CW)], tb[b], sin[b])
        pltpu.async_copy(a_hbm.at[:, pl.ds(c0, _CW)], ab[b], sin[b])

    def wait_in(k, b):
        c0 = c0_of(k)
        pltpu.make_async_copy(t_hbm.at[:, pl.ds(c0, _CW)], tb[b], sin[b]).wait()
        pltpu.make_async_copy(a_hbm.at[:, pl.ds(c0, _CW)], ab[b], sin[b]).wait()

    def start_out(k, b):
        pltpu.async_copy(ob[b], o_hbm.at[:, pl.ds(c0_of(k), _CW)], sout[b])

    def wait_out(k, b):
        pltpu.make_async_copy(
            ob[b], o_hbm.at[:, pl.ds(c0_of(k), _CW)], sout[b]).wait()

    def compute(t_buf, a_buf, o_buf):
        @plsc.parallel_loop(0, _CW // 16)
        def body(j):
            c = j * 16
            for r in range(16):
                o_buf[r, pl.ds(c, 16)] = _tok16(t_buf[r, pl.ds(c, 16)])
            for r in range(8):
                o_buf[17 + r, pl.ds(c, 16)] = _tok16(a_buf[r, pl.ds(c, 16)])
            o_buf[16, pl.ds(c, 16)] = sep

    def step(k, b):
        wait_in(k, b)

        @pl.when(k >= 2)
        def _():
            wait_out(k - 2, b)

        compute(tb[b], ab[b], ob[b])
        start_out(k, b)

        @pl.when(k + 2 < nmine)
        def _():
            start_in(k + 2, b)

    start_in(0, 0)

    @pl.when(nmine > 1)
    def _():
        start_in(1, 1)

    def pair_body(p, _):
        step(2 * p, 0)

        @pl.when(2 * p + 1 < nmine)
        def _():
            step(2 * p + 1, 1)

        return 0

    lax.fori_loop(0, (nmine + 1) // 2, pair_body, 0)
    wait_out(2 * ((nmine - 1) // 2), 0)

    @pl.when(nmine > 1)
    def _():
        wait_out(nmine - 1 - (nmine % 2), 1)

    if remw:
        @pl.when(wid == 8)
        def _():
            c0 = remc0
            pltpu.sync_copy(t_hbm.at[:, pl.ds(c0, remw)],
                            t0.at[:, pl.ds(0, remw)])
            pltpu.sync_copy(a_hbm.at[:, pl.ds(c0, remw)],
                            a0.at[:, pl.ds(0, remw)])

            @plsc.parallel_loop(0, remw // 16)
            def body(j):
                c = j * 16
                for r in range(16):
                    o0[r, pl.ds(c, 16)] = _tok16(t0[r, pl.ds(c, 16)])
                for r in range(8):
                    o0[17 + r, pl.ds(c, 16)] = _tok16(a0[r, pl.ds(c, 16)])
                o0[16, pl.ds(c, 16)] = sep

            pltpu.sync_copy(o0.at[:, pl.ds(0, remw)],
                            o_hbm.at[:, pl.ds(c0, remw)])


def _tok_ref(x):
    mu = jnp.sign(x) * jnp.log(jnp.abs(x) * _MU + 1.0) / np.log(_M * _MU + 1.0)
    v = jnp.floor((jnp.clip(mu, -1.0, 1.0) + 1.0) * (_NB / 2))
    return jnp.clip(v, 0.0, _NB - 1).astype(jnp.int32) + _SHIFT


@jax.jit
def kernel(tensors, actions):
    n = tensors.shape[0]
    nmain = (n // 128) * 128
    nfull = nmain // _CW
    remc0 = nfull * _CW
    remw = nmain - remc0
    mesh = plsc.VectorSubcoreMesh(core_axis_name="c", subcore_axis_name="s")
    run = pl.kernel(
        functools.partial(_sc_body, nfull, remc0, remw),
        out_type=jax.ShapeDtypeStruct((25, n), jnp.int32),
        mesh=mesh,
        compiler_params=pltpu.CompilerParams(needs_layout_passes=False,
                                             use_tc_tiling_on_sc=True),
        scratch_types=[
            pltpu.VMEM((16, _CW), jnp.float32),
            pltpu.VMEM((16, _CW), jnp.float32),
            pltpu.VMEM((8, _CW), jnp.float32),
            pltpu.VMEM((8, _CW), jnp.float32),
            pltpu.VMEM((25, _CW), jnp.int32),
            pltpu.VMEM((25, _CW), jnp.int32),
            pltpu.SemaphoreType.DMA,
            pltpu.SemaphoreType.DMA,
            pltpu.SemaphoreType.DMA,
            pltpu.SemaphoreType.DMA,
        ],
    )
    out = run(tensors.T, actions.T).T
    if nmain == n:
        return out
    tt = _tok_ref(tensors[nmain:])
    at = _tok_ref(actions[nmain:])
    sepcol = jnp.full((n - nmain, 1), _SEP, jnp.int32)
    tail = jnp.concatenate([tt, sepcol, at], axis=1)
    return lax.dynamic_update_slice(out, tail, (nmain, 0))


# speedup vs baseline: 10.6474x; 1.2769x over previous
import functools

import jax
import jax.numpy as jnp
import numpy as np
from jax import lax
from jax.experimental import pallas as pl
from jax.experimental.pallas import tpu as pltpu
from jax.experimental.pallas import tpu_sc as plsc

_MU = 100.0
_M = 256.0
_NB = 1024
_SHIFT = 32000
_SEP = _NB + _SHIFT
_KS = float(512.0 / np.log2(_M * _MU + 1.0))

_CW = 1024
_NSUB = 32

_IDX0 = (127 << 6)
_YCAP = float(_M * _MU + 1.0)


def _build_tables():
    idx = np.arange(1026, dtype=np.int64)
    bits = ((idx + _IDX0) << 17).astype(np.uint32)
    val = _KS * np.log2(bits.view(np.float32).astype(np.float64))
    t0 = val[:-1].astype(np.float32)[:1024]
    t1 = ((val[1:] - val[:-1]) / 2.0**17).astype(np.float32)[:1024]
    return t0, t1


_T0, _T1 = _build_tables()


def _tok16(x, c0v, c1v):
    y = jnp.abs(x) * jnp.float32(_MU) + jnp.float32(1.0)
    y = jnp.minimum(y, jnp.float32(_YCAP))
    b = plsc.bitcast(y, jnp.int32)
    i = (b >> 17) - _IDX0
    rf = (b & 0x1FFFF).astype(jnp.float32)
    c0 = plsc.load_gather(c0v, [i])
    c1 = plsc.load_gather(c1v, [i])
    l2k = c1 * rf + c0
    sgn = plsc.bitcast(x, jnp.int32) & jnp.int32(-2147483648)
    sv = plsc.bitcast(plsc.bitcast(l2k, jnp.int32) ^ sgn, jnp.float32)
    v = sv + jnp.float32(512.0 + _SHIFT)
    v = jnp.minimum(jnp.maximum(v, jnp.float32(_SHIFT)),
                    jnp.float32(_SHIFT + _NB - 1))
    return v.astype(jnp.int32)


def _sc_body(nfull, remc0, remw, t_hbm, a_hbm, c0_hbm, c1_hbm, o_hbm,
             t0, t1, a0, a1, o0, o1, c0v, c1v, si0, si1, so0, so1):
    wid = lax.axis_index("s") * 2 + lax.axis_index("c")
    sep = jnp.full((16,), _SEP, jnp.int32)
    nmine = (nfull - 1 - wid) // _NSUB + 1
    tb, ab, ob = (t0, t1), (a0, a1), (o0, o1)
    sin, sout = (si0, si1), (so0, so1)
    pltpu.sync_copy(c0_hbm, c0v)
    pltpu.sync_copy(c1_hbm, c1v)

    def c0_of(k):
        return (wid + k * _NSUB) * _CW

    def start_in(k, b):
        c0 = c0_of(k)
        pltpu.async_copy(t_hbm.at[:, pl.ds(c0, _CW)], tb[b], sin[b])
        pltpu.async_copy(a_hbm.at[:, pl.ds(c0, _CW)], ab[b], sin[b])

    def wait_in(k, b):
        c0 = c0_of(k)
        pltpu.make_async_copy(t_hbm.at[:, pl.ds(c0, _CW)], tb[b], sin[b]).wait()
        pltpu.make_async_copy(a_hbm.at[:, pl.ds(c0, _CW)], ab[b], sin[b]).wait()

    def start_out(k, b):
        pltpu.async_copy(ob[b], o_hbm.at[:, pl.ds(c0_of(k), _CW)], sout[b])

    def wait_out(k, b):
        pltpu.make_async_copy(
            ob[b], o_hbm.at[:, pl.ds(c0_of(k), _CW)], sout[b]).wait()

    def compute(t_buf, a_buf, o_buf):
        @plsc.parallel_loop(0, _CW // 16)
        def body(j):
            c = j * 16
            for r in range(16):
                o_buf[r, pl.ds(c, 16)] = _tok16(t_buf[r, pl.ds(c, 16)],
                                                c0v, c1v)
            for r in range(8):
                o_buf[17 + r, pl.ds(c, 16)] = _tok16(a_buf[r, pl.ds(c, 16)],
                                                     c0v, c1v)
            o_buf[16, pl.ds(c, 16)] = sep

    def step(k, b):
        wait_in(k, b)

        @pl.when(k >= 2)
        def _():
            wait_out(k - 2, b)

        compute(tb[b], ab[b], ob[b])
        start_out(k, b)

        @pl.when(k + 2 < nmine)
        def _():
            start_in(k + 2, b)

    start_in(0, 0)

    @pl.when(nmine > 1)
    def _():
        start_in(1, 1)

    def pair_body(p, _):
        step(2 * p, 0)

        @pl.when(2 * p + 1 < nmine)
        def _():
            step(2 * p + 1, 1)

        return 0

    lax.fori_loop(0, (nmine + 1) // 2, pair_body, 0)
    wait_out(2 * ((nmine - 1) // 2), 0)

    @pl.when(nmine > 1)
    def _():
        wait_out(nmine - 1 - (nmine % 2), 1)

    if remw:
        @pl.when(wid == 8)
        def _():
            c0 = remc0
            pltpu.sync_copy(t_hbm.at[:, pl.ds(c0, remw)],
                            t0.at[:, pl.ds(0, remw)])
            pltpu.sync_copy(a_hbm.at[:, pl.ds(c0, remw)],
                            a0.at[:, pl.ds(0, remw)])

            @plsc.parallel_loop(0, remw // 16)
            def body(j):
                c = j * 16
                for r in range(16):
                    o0[r, pl.ds(c, 16)] = _tok16(t0[r, pl.ds(c, 16)],
                                                 c0v, c1v)
                for r in range(8):
                    o0[17 + r, pl.ds(c, 16)] = _tok16(a0[r, pl.ds(c, 16)],
                                                      c0v, c1v)
                o0[16, pl.ds(c, 16)] = sep

            pltpu.sync_copy(o0.at[:, pl.ds(0, remw)],
                            o_hbm.at[:, pl.ds(c0, remw)])


def _tok_ref(x):
    mu = jnp.sign(x) * jnp.log(jnp.abs(x) * _MU + 1.0) / np.log(_M * _MU + 1.0)
    v = jnp.floor((jnp.clip(mu, -1.0, 1.0) + 1.0) * (_NB / 2))
    return jnp.clip(v, 0.0, _NB - 1).astype(jnp.int32) + _SHIFT


@jax.jit
def kernel(tensors, actions):
    n = tensors.shape[0]
    nmain = (n // 128) * 128
    nfull = nmain // _CW
    remc0 = nfull * _CW
    remw = nmain - remc0
    mesh = plsc.VectorSubcoreMesh(core_axis_name="c", subcore_axis_name="s")
    run = pl.kernel(
        functools.partial(_sc_body, nfull, remc0, remw),
        out_type=jax.ShapeDtypeStruct((25, n), jnp.int32),
        mesh=mesh,
        compiler_params=pltpu.CompilerParams(needs_layout_passes=False,
                                             use_tc_tiling_on_sc=True),
        scratch_types=[
            pltpu.VMEM((16, _CW), jnp.float32),
            pltpu.VMEM((16, _CW), jnp.float32),
            pltpu.VMEM((8, _CW), jnp.float32),
            pltpu.VMEM((8, _CW), jnp.float32),
            pltpu.VMEM((25, _CW), jnp.int32),
            pltpu.VMEM((25, _CW), jnp.int32),
            pltpu.VMEM((1024,), jnp.float32),
            pltpu.VMEM((1024,), jnp.float32),
            pltpu.SemaphoreType.DMA,
            pltpu.SemaphoreType.DMA,
            pltpu.SemaphoreType.DMA,
            pltpu.SemaphoreType.DMA,
        ],
    )
    out = run(tensors.T, actions.T, jnp.asarray(_T0), jnp.asarray(_T1)).T
    if nmain == n:
        return out
    tt = _tok_ref(tensors[nmain:])
    at = _tok_ref(actions[nmain:])
    sepcol = jnp.full((n - nmain, 1), _SEP, jnp.int32)
    tail = jnp.concatenate([tt, sepcol, at], axis=1)
    return lax.dynamic_update_slice(out, tail, (nmain, 0))
